# baseline (speedup 1.0000x reference)
import functools

import jax
import jax.numpy as jnp
from jax import lax
from jax.experimental import pallas as pl
from jax.experimental.pallas import tpu as pltpu
from jax.experimental.pallas import tpu_sc as plsc


@functools.lru_cache(maxsize=None)
def _build_gather(B: int, T: int, ROW: int):
    info = plsc.get_sparse_core_info()
    NW = info.num_cores * info.num_subcores
    b_per_w = B // NW
    CH = min(16, b_per_w)
    n_ch = b_per_w // CH
    mesh = plsc.VectorSubcoreMesh(core_axis_name="c", subcore_axis_name="s")

    NB = min(3, n_ch)

    @functools.partial(
        pl.kernel,
        mesh=mesh,
        out_type=jax.ShapeDtypeStruct((B, ROW), jnp.float32),
        scratch_types=[
            pltpu.VMEM((b_per_w,), jnp.int32),
            pltpu.VMEM((NB, CH, ROW), jnp.float32),
            pltpu.SemaphoreType.DMA,
            pltpu.SemaphoreType.DMA,
        ],
    )
    def gather_kernel(table_hbm, idx_hbm, out_hbm, idx_v, rows_v, gsem, ssem):
        wid = lax.axis_index("s") * info.num_cores + lax.axis_index("c")
        base = wid * b_per_w
        pltpu.sync_copy(idx_hbm.at[pl.ds(base, b_per_w)], idx_v)

        def fire_gather(c):
            return pltpu.async_copy(
                table_hbm.at[idx_v.at[pl.ds(c * CH, CH)]], rows_v.at[c % NB], gsem
            )

        gs = [fire_gather(c) for c in range(NB)]
        ss = [None] * n_ch
        for c in range(n_ch):
            gs[c].wait()
            ss[c] = pltpu.async_copy(
                rows_v.at[c % NB], out_hbm.at[pl.ds(base + c * CH, CH)], ssem
            )
            if c + NB < n_ch:
                ss[c].wait()
                gs.append(fire_gather(c + NB))
        for c in range(max(0, n_ch - NB), n_ch):
            if c + NB >= n_ch:
                ss[c].wait()

    return gather_kernel


def kernel(query_tokens, task_ids, batch_size):
    T, Tq, D = query_tokens.shape
    B = task_ids.shape[0]
    table = query_tokens.reshape(T, Tq * D)
    idx = task_ids.astype(jnp.int32)
    out = _build_gather(B, T, Tq * D)(table, idx)
    return out.reshape(B, Tq, D)


# speedup vs baseline: 1.6344x; 1.6344x over previous
import functools

import jax
import jax.numpy as jnp
from jax import lax
from jax.experimental import pallas as pl
from jax.experimental.pallas import tpu as pltpu
from jax.experimental.pallas import tpu_sc as plsc


@functools.lru_cache(maxsize=None)
def _build_gather(B: int, T: int, TQ: int, D: int):
    info = plsc.get_sparse_core_info()
    NW = info.num_cores * info.num_subcores
    b_per_w = B // NW
    CH = min(16, b_per_w)
    n_ch = b_per_w // CH
    NB = min(3, n_ch)
    mesh = plsc.VectorSubcoreMesh(core_axis_name="c", subcore_axis_name="s")

    @functools.partial(
        pl.kernel,
        mesh=mesh,
        out_type=jax.ShapeDtypeStruct((B, TQ, D), jnp.float32),
        scratch_types=[
            pltpu.VMEM((b_per_w,), jnp.int32),
            pltpu.VMEM((NB, CH, TQ, D), jnp.float32),
            pltpu.SemaphoreType.DMA,
            pltpu.SemaphoreType.DMA,
        ],
    )
    def gather_kernel(table_hbm, idx_hbm, out_hbm, idx_v, rows_v, gsem, ssem):
        wid = lax.axis_index("s") * info.num_cores + lax.axis_index("c")
        base = wid * b_per_w
        pltpu.sync_copy(idx_hbm.at[pl.ds(base, b_per_w)], idx_v)

        def fire_gather(c):
            return pltpu.async_copy(
                table_hbm.at[idx_v.at[pl.ds(c * CH, CH)]], rows_v.at[c % NB], gsem
            )

        gs = [fire_gather(c) for c in range(NB)]
        ss = [None] * n_ch
        for c in range(n_ch):
            gs[c].wait()
            ss[c] = pltpu.async_copy(
                rows_v.at[c % NB], out_hbm.at[pl.ds(base + c * CH, CH)], ssem
            )
            if c + NB < n_ch:
                ss[c].wait()
                gs.append(fire_gather(c + NB))
        for c in range(max(0, n_ch - NB), n_ch):
            if c + NB >= n_ch:
                ss[c].wait()

    return gather_kernel


def kernel(query_tokens, task_ids, batch_size):
    T, Tq, D = query_tokens.shape
    B = task_ids.shape[0]
    idx = task_ids.astype(jnp.int32)
    return _build_gather(B, T, Tq, D)(query_tokens, idx)


# speedup vs baseline: 2.4001x; 1.4685x over previous
import functools

import jax
import jax.numpy as jnp
from jax import lax
from jax.experimental import pallas as pl
from jax.experimental.pallas import tpu as pltpu
from jax.experimental.pallas import tpu_sc as plsc


@functools.lru_cache(maxsize=None)
def _build_gather(B: int, T: int, TQ: int, D: int):
    info = plsc.get_sparse_core_info()
    NW = info.num_cores * info.num_subcores
    b_per_w = B // NW
    CH = min(16, b_per_w)
    n_ch = b_per_w // CH
    NB = min(3, n_ch)
    mesh = plsc.VectorSubcoreMesh(core_axis_name="c", subcore_axis_name="s")

    @functools.partial(
        pl.kernel,
        mesh=mesh,
        out_type=jax.ShapeDtypeStruct((B, TQ, D), jnp.float32),
        scratch_types=[
            pltpu.VMEM((b_per_w,), jnp.int32),
            pltpu.VMEM((NB, CH, TQ, D), jnp.float32),
            pltpu.VMEM_SHARED((T, TQ, D), jnp.float32),
            pltpu.SemaphoreType.DMA,
            pltpu.SemaphoreType.DMA,
        ],
    )
    def gather_kernel(table_hbm, idx_hbm, out_hbm, idx_v, rows_v, table_sh, gsem, ssem):
        sid = lax.axis_index("s")
        wid = sid * info.num_cores + lax.axis_index("c")
        base = wid * b_per_w
        @pl.when(sid == 0)
        def _stage():
            pltpu.sync_copy(table_hbm, table_sh)

        pltpu.sync_copy(idx_hbm.at[pl.ds(base, b_per_w)], idx_v)
        plsc.subcore_barrier()

        def fire_gather(c):
            return pltpu.async_copy(
                table_sh.at[idx_v.at[pl.ds(c * CH, CH)]], rows_v.at[c % NB], gsem
            )

        gs = [fire_gather(c) for c in range(NB)]
        ss = [None] * n_ch
        for c in range(n_ch):
            gs[c].wait()
            ss[c] = pltpu.async_copy(
                rows_v.at[c % NB], out_hbm.at[pl.ds(base + c * CH, CH)], ssem
            )
            if c + NB < n_ch:
                ss[c].wait()
                gs.append(fire_gather(c + NB))
        for c in range(max(0, n_ch - NB), n_ch):
            if c + NB >= n_ch:
                ss[c].wait()

    return gather_kernel


def kernel(query_tokens, task_ids, batch_size):
    T, Tq, D = query_tokens.shape
    B = task_ids.shape[0]
    idx = task_ids.astype(jnp.int32)
    return _build_gather(B, T, Tq, D)(query_tokens, idx)
